# baseline (speedup 1.0000x reference)
import functools

import jax
import jax.numpy as jnp
from jax import lax
from jax.experimental import pallas as pl
from jax.experimental.pallas import tpu as pltpu
from jax.experimental.pallas import tpu_sc as plsc

B = 16384
D_BLK = 64
D_DIR = 32
NC, NS = 2, 16
NW = NC * NS
BPW = B // NW
NGRP = BPW // 16
NCK = 4
GPC = NGRP // NCK
CROWS = BPW // NCK

_mesh = plsc.VectorSubcoreMesh(core_axis_name="c", subcore_axis_name="s")


@functools.partial(
    pl.kernel,
    out_type=jax.ShapeDtypeStruct((B, 128), jnp.float32),
    mesh=_mesh,
    compiler_params=pltpu.CompilerParams(needs_layout_passes=False),
    scratch_types=[
        pltpu.VMEM((BPW,), jnp.int32),
        pltpu.VMEM((2, 16, 8, D_BLK), jnp.float32),
        pltpu.VMEM((1002 * D_DIR,), jnp.float32),
        pltpu.VMEM((CROWS, 128), jnp.float32),
        pltpu.SemaphoreType.DMA,
        pltpu.SemaphoreType.DMA,
    ],
)
def _encode(comb_idx_hbm, dir_tab_hbm, blk_tab_hbm, out_hbm,
            idx_v, oct_v, dir_all_v, out_c, sem_b, sem_d):
    wid = lax.axis_index("s") * NC + lax.axis_index("c")
    base = wid * BPW
    pltpu.sync_copy(comb_idx_hbm.at[pl.ds(base, BPW)], idx_v)
    pltpu.sync_copy(dir_tab_hbm, dir_all_v)
    lanes = lax.broadcasted_iota(jnp.int32, (16,), 0)
    zeros = jnp.zeros((16,), jnp.int32)

    def scalars(g):
        cvec = idx_v[pl.ds(g * 16, 16)]
        out = []
        for l in range(16):
            s = lax.reduce_max(jnp.where(lanes == l, cvec, zeros), axes=(0,))
            bi = s // 1024
            di = s - bi * 1024
            out.append((bi, di))
        return out

    def issue(g):
        slot = lax.rem(g, 2)
        for l, (bi, di) in enumerate(scalars(g)):
            bo = (bi // 8) * 8
            pltpu.async_copy(
                blk_tab_hbm.at[pl.ds(bo, 8)], oct_v.at[slot, l], sem_b)

    def drain_extract(g):
        slot = lax.rem(g, 2)
        for l in range(16):
            pltpu.make_async_copy(
                blk_tab_hbm.at[pl.ds(0, 8)], oct_v.at[slot, l], sem_b).wait()
        row0 = lax.rem(g, GPC) * 16
        for l, (bi, di) in enumerate(scalars(g)):
            br = lax.rem(bi, 8)
            row = row0 + l
            for k in range(4):
                out_c[row, pl.ds(k * 16, 16)] = oct_v[slot, l, br,
                                                      pl.ds(k * 16, 16)]
            for k in range(2):
                out_c[row, pl.ds(D_BLK + k * 16, 16)] = dir_all_v[
                    pl.ds(di * D_DIR + k * 16, 16)]

    issue(0)

    def step(g, carry):
        @pl.when(g < NGRP - 1)
        def _():
            issue(g + 1)
        drain_extract(g)

        @pl.when(lax.rem(g, GPC) == GPC - 1)
        def _():
            c = g // GPC
            pltpu.sync_copy(out_c, out_hbm.at[pl.ds(base + c * CROWS, CROWS)])
        return carry

    lax.fori_loop(0, NGRP, step, 0)


def kernel(direction_batch, block_batch, direction_table, block_table):
    comb = block_batch.reshape(B) * 1024 + direction_batch.reshape(B)
    out = _encode(comb, direction_table.reshape(-1), block_table)
    return out[:, :D_BLK + D_DIR]
